# Initial kernel scaffold; baseline (speedup 1.0000x reference)
import jax
import jax.numpy as jnp
from jax.experimental import pallas as pl


def kernel(hidden_states, temperature, top_p, embd_weight):
    raise NotImplementedError("write your pallas kernel here")


# speedup vs baseline: 16.4198x; 16.4198x over previous
import functools

import jax
import jax.numpy as jnp
import numpy as np
from jax import lax
from jax.experimental import pallas as pl
from jax.experimental.pallas import tpu as pltpu
from jax.experimental.pallas import tpu_sc as plsc

B = 32
D = 128
V = 1000000
NB = 4096
NBF = np.float32(NB)
VT1 = 8192
VT3 = 8192
CHUNK = 20000
TINY = np.float32(np.finfo(np.float32).tiny)
NEG_EPS = np.float32(1e-38)


def _mm_body(h_ref, t_ref, w_ref, lg_ref, mn_ref, mx_ref, rmin_ref, rmax_ref,
             *, nblk, v_len):
    i = pl.program_id(0)
    blk = lax.dot_general(h_ref[...], w_ref[...], (((1,), (1,)), ((), ())),
                          preferred_element_type=jnp.float32)
    lt = blk / t_ref[...]
    lg_ref[...] = lt
    col = lax.broadcasted_iota(jnp.int32, lt.shape, 1) + i * VT1
    valid = col < v_len
    bmin = jnp.min(jnp.where(valid, lt, jnp.inf), axis=1, keepdims=True)
    bmax = jnp.max(jnp.where(valid, lt, -jnp.inf), axis=1, keepdims=True)
    bmin_b = jnp.broadcast_to(bmin, (B, 128))
    bmax_b = jnp.broadcast_to(bmax, (B, 128))

    @pl.when(i == 0)
    def _():
        rmin_ref[...] = bmin_b
        rmax_ref[...] = bmax_b

    @pl.when(i > 0)
    def _():
        rmin_ref[...] = jnp.minimum(rmin_ref[...], bmin_b)
        rmax_ref[...] = jnp.maximum(rmax_ref[...], bmax_b)

    @pl.when(i == nblk - 1)
    def _():
        mn_ref[...] = jnp.min(rmin_ref[...], axis=1, keepdims=True)
        mx_ref[...] = jnp.max(rmax_ref[...], axis=1, keepdims=True)


def _stage1(hidden, temp2, w):
    nblk = (V + VT1 - 1) // VT1
    return pl.pallas_call(
        functools.partial(_mm_body, nblk=nblk, v_len=V),
        grid=(nblk,),
        in_specs=[
            pl.BlockSpec((B, D), lambda i: (0, 0)),
            pl.BlockSpec((B, 1), lambda i: (0, 0)),
            pl.BlockSpec((VT1, D), lambda i: (i, 0)),
        ],
        out_specs=[
            pl.BlockSpec((B, VT1), lambda i: (0, i)),
            pl.BlockSpec((B, 1), lambda i: (0, 0)),
            pl.BlockSpec((B, 1), lambda i: (0, 0)),
        ],
        out_shape=[
            jax.ShapeDtypeStruct((B, V), jnp.float32),
            jax.ShapeDtypeStruct((B, 1), jnp.float32),
            jax.ShapeDtypeStruct((B, 1), jnp.float32),
        ],
        scratch_shapes=[
            pltpu.VMEM((B, 128), jnp.float32),
            pltpu.VMEM((B, 128), jnp.float32),
        ],
    )(hidden, temp2, w)


def _lane_scalar(vec, lane):
    sel = jnp.where(lax.iota(jnp.int32, 16) == lane, vec, -jnp.inf)
    return jnp.max(sel)


def _sc_body(lg_hbm, mn_hbm, mx_hbm, tp_hbm, s1_hbm, w1_hbm, s2_hbm, out_hbm,
             buf, hist, merged, mn_v, mx_v, tp_v, s1_v, w1_v, s2_v, outbuf,
             *, v_len):
    nch = v_len // CHUNK
    nvr = CHUNK // 16
    wid = lax.axis_index("s") * 2 + lax.axis_index("c")
    r = wid
    pltpu.sync_copy(mn_hbm, mn_v)
    pltpu.sync_copy(mx_hbm, mx_v)
    pltpu.sync_copy(tp_hbm, tp_v)
    pltpu.sync_copy(s1_hbm, s1_v)
    pltpu.sync_copy(w1_hbm, w1_v)
    pltpu.sync_copy(s2_hbm, s2_v)
    cbase = (r // 16) * 16
    lane = r % 16
    m_s = _lane_scalar(mn_v[pl.ds(cbase, 16)], lane)
    M_s = _lane_scalar(mx_v[pl.ds(cbase, 16)], lane)
    tp_s = _lane_scalar(tp_v[pl.ds(cbase, 16)], lane)
    s1_s = _lane_scalar(s1_v[pl.ds(cbase, 16)], lane)
    w1_s = _lane_scalar(w1_v[pl.ds(cbase, 16)], lane)
    s2_s = _lane_scalar(s2_v[pl.ds(cbase, 16)], lane)
    mb = jnp.full((16,), m_s, jnp.float32)
    Mb = jnp.full((16,), M_s, jnp.float32)
    s1b = jnp.full((16,), s1_s, jnp.float32)
    lanebase = lax.iota(jnp.int32, 16) * NB

    def zero_hist():
        def zloop(j, c):
            hist[pl.ds(j * 16, 16)] = jnp.zeros((16,), jnp.float32)
            return c
        lax.fori_loop(0, (16 * NB) // 16, zloop, 0)

    def merge_total():
        def mloop(cb, tot):
            acc = jnp.zeros((16,), jnp.float32)
            for l in range(16):
                acc = acc + hist[pl.ds(l * NB + cb * 16, 16)]
            merged[pl.ds(cb * 16, 16)] = acc
            return tot + jnp.sum(acc)
        return lax.fori_loop(0, NB // 16, mloop, jnp.float32(0.0))

    def walk(tpz, offset):
        def wloop(t, carry):
            found, bstar, sab, sinc, csum = carry
            cb = NB // 16 - 1 - t
            vv = merged[pl.ds(cb * 16, 16)]
            tot = jnp.sum(vv)
            pre = plsc.cumsum(vv)
            sufinc = offset + (csum + (tot - pre) + vv)
            maskv = sufinc > tpz
            cnt = jnp.sum(maskv.astype(jnp.int32))
            has = cnt > 0
            first = jnp.logical_and(has, jnp.logical_not(found))
            blocal = cnt - 1
            pre_at = _lane_scalar(pre, blocal)
            v_at = _lane_scalar(vv, blocal)
            sab_new = csum + (tot - pre_at)
            sinc_new = offset + (sab_new + v_at)
            return (jnp.logical_or(found, has),
                    jnp.where(first, cb * 16 + blocal, bstar),
                    jnp.where(first, sab_new, sab),
                    jnp.where(first, sinc_new, sinc),
                    csum + tot)
        init = (jnp.bool_(False), jnp.int32(0), jnp.float32(0.0),
                jnp.float32(1.0), jnp.float32(0.0))
        found, bstar, sab, sinc, _ = lax.fori_loop(0, NB // 16, wloop, init)
        return bstar, sab, sinc

    def stream(pass2, b1_s, lo2b, s2b):
        def chunk_loop(c, _):
            pltpu.sync_copy(lg_hbm.at[r, pl.ds(c * CHUNK, CHUNK)], buf)

            def vloop(k, _unused):
                vv = buf[pl.ds(k * 16, 16)]
                e = jnp.exp(vv - Mb)
                t1 = (vv - mb) * s1b
                b1v = jnp.minimum(t1.astype(jnp.int32), NB - 1)
                if not pass2:
                    idx = lanebase + b1v
                    plsc.addupdate_scatter(hist, [idx], e)
                else:
                    t2 = (vv - lo2b) * s2b
                    b2v = jnp.clip(t2.astype(jnp.int32), 0, NB - 1)
                    idx = lanebase + b2v
                    selm = b1v == jnp.full((16,), b1_s, jnp.int32)
                    plsc.addupdate_scatter(hist, [idx], e, mask=selm)
                return 0
            lax.fori_loop(0, nvr, vloop, 0)
            return 0
        lax.fori_loop(0, nch, chunk_loop, 0)

    zero_hist()
    stream(False, None, None, None)
    z_tot = merge_total()
    tpz = tp_s * z_tot
    b1, sab1, _ = walk(tpz, jnp.float32(0.0))

    b1f = b1.astype(jnp.float32)
    lo2_s = m_s + b1f * w1_s
    lo2b = jnp.full((16,), lo2_s, jnp.float32)
    s2b = jnp.full((16,), s2_s, jnp.float32)
    zero_hist()
    stream(True, b1, lo2b, s2b)
    merge_total()
    b2, _, z2 = walk(tpz, sab1)

    io16 = lax.iota(jnp.int32, 16)
    ov = jnp.where(io16 == 0, b1f,
                   jnp.where(io16 == 1, b2.astype(jnp.float32),
                             jnp.where(io16 == 2, z2, jnp.float32(0.0))))
    outbuf[...] = ov
    pltpu.sync_copy(outbuf, out_hbm.at[r])


def _stage2(logits, mn, mx, top_p, s1, w1, s2, v_len=V):
    mesh = plsc.VectorSubcoreMesh(core_axis_name="c", subcore_axis_name="s")
    kern = pl.kernel(
        functools.partial(_sc_body, v_len=v_len),
        out_type=jax.ShapeDtypeStruct((B, 16), jnp.float32),
        mesh=mesh,
        scratch_types=[
            pltpu.VMEM((CHUNK,), jnp.float32),
            pltpu.VMEM((16 * NB,), jnp.float32),
            pltpu.VMEM((NB,), jnp.float32),
            pltpu.VMEM((B,), jnp.float32),
            pltpu.VMEM((B,), jnp.float32),
            pltpu.VMEM((B,), jnp.float32),
            pltpu.VMEM((B,), jnp.float32),
            pltpu.VMEM((B,), jnp.float32),
            pltpu.VMEM((B,), jnp.float32),
            pltpu.VMEM((16,), jnp.float32),
        ],
        compiler_params=pltpu.CompilerParams(use_tc_tiling_on_sc=False,
                                             needs_layout_passes=False),
    )
    return kern(logits, mn, mx, top_p, s1, w1, s2)


_ROT = ((13, 15, 26, 6), (17, 29, 16, 24))


def _threefry_bits(j):
    k0 = jnp.uint32(0)
    k1 = jnp.uint32(42)
    k2 = jnp.uint32(0 ^ 42 ^ 0x1BD11BDA)
    ks = (k0, k1, k2)
    x0 = jnp.zeros_like(j) + ks[0]
    x1 = j + ks[1]
    for g in range(5):
        for rr in _ROT[g % 2]:
            x0 = x0 + x1
            x1 = (x1 << jnp.uint32(rr)) | (x1 >> jnp.uint32(32 - rr))
            x1 = x0 ^ x1
        x0 = x0 + ks[(g + 1) % 3]
        x1 = x1 + ks[(g + 2) % 3] + jnp.uint32(g + 1)
    return x0 ^ x1


def _smp_body(lg_ref, mn_ref, mx_ref, s1_ref, w1_ref, s2_ref,
              b1_ref, b2_ref, z2_ref, ids_ref,
              bv_ref, bi_ref, *, nblk, v_len):
    i = pl.program_id(0)
    lt = lg_ref[...]
    mnb = mn_ref[...]
    mxb = mx_ref[...]
    t1 = (lt - mnb) * s1_ref[...]
    bin1 = jnp.minimum(t1.astype(jnp.int32), NB - 1)
    b1f = b1_ref[...]
    b1i = b1f.astype(jnp.int32)
    lo2 = mnb + b1f * w1_ref[...]
    t2 = (lt - lo2) * s2_ref[...]
    bin2 = jnp.clip(t2.astype(jnp.int32), 0, NB - 1)
    b2i = b2_ref[...].astype(jnp.int32)
    kept = (bin1 > b1i) | ((bin1 == b1i) & (bin2 >= b2i))
    e = jnp.exp(lt - mxb)
    p2 = jnp.where(kept, e / z2_ref[...], jnp.float32(0.0))
    z = jnp.log(p2 + NEG_EPS)

    col = lax.broadcasted_iota(jnp.int32, lt.shape, 1) + i * VT3
    row = lax.broadcasted_iota(jnp.int32, lt.shape, 0)
    j = (row * v_len + col).astype(jnp.uint32)
    bits = _threefry_bits(j)
    fb = (bits >> jnp.uint32(9)) | jnp.uint32(0x3F800000)
    f = lax.bitcast_convert_type(fb, jnp.float32) - jnp.float32(1.0)
    u = jnp.maximum(TINY, f * jnp.float32(1.0) + TINY)
    g = -jnp.log(-jnp.log(u))

    s = jnp.where(col < v_len, g + z, -jnp.inf)
    bmax = jnp.max(s, axis=1, keepdims=True)
    cand = jnp.where(s == bmax, col, jnp.int32(2**31 - 1))
    bidx = jnp.min(cand, axis=1, keepdims=True)
    bmax_b = jnp.broadcast_to(bmax, (B, 128))
    bidx_b = jnp.broadcast_to(bidx, (B, 128))

    @pl.when(i == 0)
    def _():
        bv_ref[...] = bmax_b
        bi_ref[...] = bidx_b

    @pl.when(i > 0)
    def _():
        upd = bmax_b > bv_ref[...]
        bv_ref[...] = jnp.where(upd, bmax_b, bv_ref[...])
        bi_ref[...] = jnp.where(upd, bidx_b, bi_ref[...])

    @pl.when(i == nblk - 1)
    def _():
        ids_ref[...] = jnp.min(bi_ref[...], axis=1, keepdims=True)


def _stage3(logits, mn, mx, s1, w1, s2, b1f, b2f, z2, v_len=V):
    nblk = (v_len + VT3 - 1) // VT3
    return pl.pallas_call(
        functools.partial(_smp_body, nblk=nblk, v_len=v_len),
        grid=(nblk,),
        in_specs=[pl.BlockSpec((B, VT3), lambda i: (0, i))] + [
            pl.BlockSpec((B, 1), lambda i: (0, 0)) for _ in range(8)],
        out_specs=pl.BlockSpec((B, 1), lambda i: (0, 0)),
        out_shape=jax.ShapeDtypeStruct((B, 1), jnp.int32),
        scratch_shapes=[
            pltpu.VMEM((B, 128), jnp.float32),
            pltpu.VMEM((B, 128), jnp.int32),
        ],
    )(logits, mn, mx, s1, w1, s2, b1f, b2f, z2)


def kernel(hidden_states, temperature, top_p, embd_weight):
    temp2 = temperature.reshape(B, 1)
    logits, mn, mx = _stage1(hidden_states, temp2, embd_weight)
    s1 = NBF / (mx - mn)
    w1 = (mx - mn) / NBF
    s2 = NBF / w1
    params = _stage2(logits, mn.reshape(B), mx.reshape(B), top_p,
                     s1.reshape(B), w1.reshape(B), s2.reshape(B))
    b1f = params[:, 0:1]
    b2f = params[:, 1:2]
    z2 = params[:, 2:3]
    ids2 = _stage3(logits, mn, mx, s1, w1, s2, b1f, b2f, z2)
    return ids2.reshape(B)


# speedup vs baseline: 29.1342x; 1.7743x over previous
import functools

import jax
import jax.numpy as jnp
import numpy as np
from jax import lax
from jax.experimental import pallas as pl
from jax.experimental.pallas import tpu as pltpu
from jax.experimental.pallas import tpu_sc as plsc

B = 32
D = 128
V = 1000000
NB = 4096
NBF = np.float32(NB)
VT1 = 8192
VT3 = 8192
CBR = 96
TINY = np.float32(np.finfo(np.float32).tiny)
NEG_EPS = np.float32(1e-38)


def _mm_body(h_ref, t_ref, w_ref, lg_ref, mn_ref, mx_ref, rmin_ref, rmax_ref,
             *, nblk, v_len):
    i = pl.program_id(0)
    blk = lax.dot_general(h_ref[...], w_ref[...], (((1,), (1,)), ((), ())),
                          preferred_element_type=jnp.float32)
    lt = blk / t_ref[...]
    col = lax.broadcasted_iota(jnp.int32, lt.shape, 1) + i * VT1
    valid = col < v_len
    lt = jnp.where(valid, lt, -jnp.inf)
    lg_ref[...] = lt.reshape(B, VT1 // 128, 128)
    bmin = jnp.min(jnp.where(valid, lt, jnp.inf), axis=1, keepdims=True)
    bmax = jnp.max(lt, axis=1, keepdims=True)
    bmin_b = jnp.broadcast_to(bmin, (B, 128))
    bmax_b = jnp.broadcast_to(bmax, (B, 128))

    @pl.when(i == 0)
    def _():
        rmin_ref[...] = bmin_b
        rmax_ref[...] = bmax_b

    @pl.when(i > 0)
    def _():
        rmin_ref[...] = jnp.minimum(rmin_ref[...], bmin_b)
        rmax_ref[...] = jnp.maximum(rmax_ref[...], bmax_b)

    @pl.when(i == nblk - 1)
    def _():
        mn_ref[...] = jnp.min(rmin_ref[...], axis=1, keepdims=True)
        mx_ref[...] = jnp.max(rmax_ref[...], axis=1, keepdims=True)


def _stage1(hidden, temp2, w, v_real=V):
    nblk = (v_real + VT1 - 1) // VT1
    vb = nblk * (VT1 // 128)
    return pl.pallas_call(
        functools.partial(_mm_body, nblk=nblk, v_len=v_real),
        grid=(nblk,),
        in_specs=[
            pl.BlockSpec((B, D), lambda i: (0, 0)),
            pl.BlockSpec((B, 1), lambda i: (0, 0)),
            pl.BlockSpec((VT1, D), lambda i: (i, 0)),
        ],
        out_specs=[
            pl.BlockSpec((B, VT1 // 128, 128), lambda i: (0, i, 0)),
            pl.BlockSpec((B, 1), lambda i: (0, 0)),
            pl.BlockSpec((B, 1), lambda i: (0, 0)),
        ],
        out_shape=[
            jax.ShapeDtypeStruct((B, vb, 128), jnp.float32),
            jax.ShapeDtypeStruct((B, 1), jnp.float32),
            jax.ShapeDtypeStruct((B, 1), jnp.float32),
        ],
        scratch_shapes=[
            pltpu.VMEM((B, 128), jnp.float32),
            pltpu.VMEM((B, 128), jnp.float32),
        ],
    )(hidden, temp2, w)


def _lane_scalar(vec, lane):
    sel = jnp.where(lax.iota(jnp.int32, 16) == lane, vec, -jnp.inf)
    return jnp.max(sel)


def _sc_body(lg_hbm, mn_hbm, mx_hbm, tp_hbm, s1_hbm, w1_hbm, s2_hbm, out_hbm,
             buf, hist, merged, mn_v, mx_v, tp_v, s1_v, w1_v, s2_v, outbuf,
             *, vb):
    nch = vb // CBR
    wid = lax.axis_index("s") * 2 + lax.axis_index("c")
    r = wid
    pltpu.sync_copy(mn_hbm, mn_v)
    pltpu.sync_copy(mx_hbm, mx_v)
    pltpu.sync_copy(tp_hbm, tp_v)
    pltpu.sync_copy(s1_hbm, s1_v)
    pltpu.sync_copy(w1_hbm, w1_v)
    pltpu.sync_copy(s2_hbm, s2_v)
    cbase = (r // 16) * 16
    lane = r % 16
    m_s = _lane_scalar(mn_v[pl.ds(cbase, 16)], lane)
    M_s = _lane_scalar(mx_v[pl.ds(cbase, 16)], lane)
    tp_s = _lane_scalar(tp_v[pl.ds(cbase, 16)], lane)
    s1_s = _lane_scalar(s1_v[pl.ds(cbase, 16)], lane)
    w1_s = _lane_scalar(w1_v[pl.ds(cbase, 16)], lane)
    s2_s = _lane_scalar(s2_v[pl.ds(cbase, 16)], lane)
    mb = jnp.full((16,), m_s, jnp.float32)
    Mb = jnp.full((16,), M_s, jnp.float32)
    s1b = jnp.full((16,), s1_s, jnp.float32)
    lanebase = lax.iota(jnp.int32, 16) * NB

    def zero_hist():
        def zloop(j, c):
            hist[pl.ds(j * 16, 16)] = jnp.zeros((16,), jnp.float32)
            return c
        lax.fori_loop(0, (16 * NB) // 16, zloop, 0)

    def merge_total():
        def mloop(cb, tot):
            acc = jnp.zeros((16,), jnp.float32)
            for l in range(16):
                acc = acc + hist[pl.ds(l * NB + cb * 16, 16)]
            merged[pl.ds(cb * 16, 16)] = acc
            return tot + jnp.sum(acc)
        return lax.fori_loop(0, NB // 16, mloop, jnp.float32(0.0))

    def walk(tpz, offset):
        def wloop(t, carry):
            found, bstar, sab, sinc, csum = carry
            cb = NB // 16 - 1 - t
            vv = merged[pl.ds(cb * 16, 16)]
            tot = jnp.sum(vv)
            pre = plsc.cumsum(vv)
            sufinc = offset + (csum + (tot - pre) + vv)
            maskv = sufinc > tpz
            cnt = jnp.sum(maskv.astype(jnp.int32))
            has = cnt > 0
            first = jnp.logical_and(has, jnp.logical_not(found))
            blocal = cnt - 1
            pre_at = _lane_scalar(pre, blocal)
            v_at = _lane_scalar(vv, blocal)
            sab_new = csum + (tot - pre_at)
            sinc_new = offset + (sab_new + v_at)
            return (jnp.logical_or(found, has),
                    jnp.where(first, cb * 16 + blocal, bstar),
                    jnp.where(first, sab_new, sab),
                    jnp.where(first, sinc_new, sinc),
                    csum + tot)
        init = (jnp.bool_(False), jnp.int32(0), jnp.float32(0.0),
                jnp.float32(1.0), jnp.float32(0.0))
        found, bstar, sab, sinc, _ = lax.fori_loop(0, NB // 16, wloop, init)
        return bstar, sab, sinc

    def stream(pass2, b1_s, lo2b, s2b):
        def chunk_loop(c, _):
            pltpu.sync_copy(lg_hbm.at[r, pl.ds(c * CBR, CBR)], buf)

            def vloop(rr, _unused):
                for u in range(8):
                    vv = buf[rr, pl.ds(u * 16, 16)]
                    e = jnp.exp(vv - Mb)
                    t1 = (vv - mb) * s1b
                    b1v = jnp.clip(t1.astype(jnp.int32), 0, NB - 1)
                    if not pass2:
                        idx = lanebase + b1v
                        plsc.addupdate_scatter(hist, [idx], e)
                    else:
                        t2 = (vv - lo2b) * s2b
                        b2v = jnp.clip(t2.astype(jnp.int32), 0, NB - 1)
                        idx = lanebase + b2v
                        selm = b1v == jnp.full((16,), b1_s, jnp.int32)
                        plsc.addupdate_scatter(hist, [idx], e, mask=selm)
                return 0
            lax.fori_loop(0, CBR, vloop, 0)
            return 0
        lax.fori_loop(0, nch, chunk_loop, 0)

    zero_hist()
    stream(False, None, None, None)
    z_tot = merge_total()
    tpz = tp_s * z_tot
    b1, sab1, _ = walk(tpz, jnp.float32(0.0))

    b1f = b1.astype(jnp.float32)
    lo2_s = m_s + b1f * w1_s
    lo2b = jnp.full((16,), lo2_s, jnp.float32)
    s2b = jnp.full((16,), s2_s, jnp.float32)
    zero_hist()
    stream(True, b1, lo2b, s2b)
    merge_total()
    b2, _, z2 = walk(tpz, sab1)

    io16 = lax.iota(jnp.int32, 16)
    ov = jnp.where(io16 == 0, b1f,
                   jnp.where(io16 == 1, b2.astype(jnp.float32),
                             jnp.where(io16 == 2, z2, jnp.float32(0.0))))
    outbuf[0, pl.ds(0, 16)] = ov
    pltpu.sync_copy(outbuf, out_hbm.at[r])


def _stage2(logits3d, mn, mx, top_p, s1, w1, s2):
    vb = logits3d.shape[1]
    mesh = plsc.VectorSubcoreMesh(core_axis_name="c", subcore_axis_name="s")
    kern = pl.kernel(
        functools.partial(_sc_body, vb=vb),
        out_type=jax.ShapeDtypeStruct((B, 1, 16), jnp.float32),
        mesh=mesh,
        scratch_types=[
            pltpu.VMEM((CBR, 128), jnp.float32),
            pltpu.VMEM((16 * NB,), jnp.float32),
            pltpu.VMEM((NB,), jnp.float32),
            pltpu.VMEM((B,), jnp.float32),
            pltpu.VMEM((B,), jnp.float32),
            pltpu.VMEM((B,), jnp.float32),
            pltpu.VMEM((B,), jnp.float32),
            pltpu.VMEM((B,), jnp.float32),
            pltpu.VMEM((B,), jnp.float32),
            pltpu.VMEM((1, 16), jnp.float32),
        ],
        compiler_params=pltpu.CompilerParams(needs_layout_passes=False),
    )
    return kern(logits3d, mn, mx, top_p, s1, w1, s2)


_ROT = ((13, 15, 26, 6), (17, 29, 16, 24))


def _threefry_bits(j):
    k0 = jnp.uint32(0)
    k1 = jnp.uint32(42)
    k2 = jnp.uint32(0 ^ 42 ^ 0x1BD11BDA)
    ks = (k0, k1, k2)
    x0 = jnp.zeros_like(j) + ks[0]
    x1 = j + ks[1]
    for g in range(5):
        for rr in _ROT[g % 2]:
            x0 = x0 + x1
            x1 = (x1 << jnp.uint32(rr)) | (x1 >> jnp.uint32(32 - rr))
            x1 = x0 ^ x1
        x0 = x0 + ks[(g + 1) % 3]
        x1 = x1 + ks[(g + 2) % 3] + jnp.uint32(g + 1)
    return x0 ^ x1


def _smp_body(lg_ref, mn_ref, mx_ref, s1_ref, w1_ref, s2_ref,
              b1_ref, b2_ref, z2_ref, ids_ref,
              bv_ref, bi_ref, *, nblk, v_len):
    i = pl.program_id(0)
    lt = lg_ref[...].reshape(B, VT3)
    mnb = mn_ref[...]
    mxb = mx_ref[...]
    t1 = (lt - mnb) * s1_ref[...]
    bin1 = jnp.minimum(t1.astype(jnp.int32), NB - 1)
    b1f = b1_ref[...]
    b1i = b1f.astype(jnp.int32)
    lo2 = mnb + b1f * w1_ref[...]
    t2 = (lt - lo2) * s2_ref[...]
    bin2 = jnp.clip(t2.astype(jnp.int32), 0, NB - 1)
    b2i = b2_ref[...].astype(jnp.int32)
    kept = (bin1 > b1i) | ((bin1 == b1i) & (bin2 >= b2i))
    e = jnp.exp(lt - mxb)
    p2 = jnp.where(kept, e / z2_ref[...], jnp.float32(0.0))
    z = jnp.log(p2 + NEG_EPS)

    col = lax.broadcasted_iota(jnp.int32, lt.shape, 1) + i * VT3
    row = lax.broadcasted_iota(jnp.int32, lt.shape, 0)
    j = (row * v_len + col).astype(jnp.uint32)
    bits = _threefry_bits(j)
    fb = (bits >> jnp.uint32(9)) | jnp.uint32(0x3F800000)
    f = lax.bitcast_convert_type(fb, jnp.float32) - jnp.float32(1.0)
    u = jnp.maximum(TINY, f * jnp.float32(1.0) + TINY)
    g = -jnp.log(-jnp.log(u))

    s = jnp.where(col < v_len, g + z, -jnp.inf)
    bmax = jnp.max(s, axis=1, keepdims=True)
    cand = jnp.where(s == bmax, col, jnp.int32(2**31 - 1))
    bidx = jnp.min(cand, axis=1, keepdims=True)
    bmax_b = jnp.broadcast_to(bmax, (B, 128))
    bidx_b = jnp.broadcast_to(bidx, (B, 128))

    @pl.when(i == 0)
    def _():
        bv_ref[...] = bmax_b
        bi_ref[...] = bidx_b

    @pl.when(i > 0)
    def _():
        upd = bmax_b > bv_ref[...]
        bv_ref[...] = jnp.where(upd, bmax_b, bv_ref[...])
        bi_ref[...] = jnp.where(upd, bidx_b, bi_ref[...])

    @pl.when(i == nblk - 1)
    def _():
        ids_ref[...] = jnp.min(bi_ref[...], axis=1, keepdims=True)


def _stage3(logits3d, mn, mx, s1, w1, s2, b1f, b2f, z2, v_real=V):
    nblk = logits3d.shape[1] // (VT3 // 128)
    return pl.pallas_call(
        functools.partial(_smp_body, nblk=nblk, v_len=v_real),
        grid=(nblk,),
        in_specs=[pl.BlockSpec((B, VT3 // 128, 128), lambda i: (0, i, 0))] + [
            pl.BlockSpec((B, 1), lambda i: (0, 0)) for _ in range(8)],
        out_specs=pl.BlockSpec((B, 1), lambda i: (0, 0)),
        out_shape=jax.ShapeDtypeStruct((B, 1), jnp.int32),
        scratch_shapes=[
            pltpu.VMEM((B, 128), jnp.float32),
            pltpu.VMEM((B, 128), jnp.int32),
        ],
    )(logits3d, mn, mx, s1, w1, s2, b1f, b2f, z2)


def kernel(hidden_states, temperature, top_p, embd_weight):
    temp2 = temperature.reshape(B, 1)
    logits3d, mn, mx = _stage1(hidden_states, temp2, embd_weight)
    s1 = NBF / (mx - mn)
    w1 = (mx - mn) / NBF
    s2 = NBF / w1
    params = _stage2(logits3d, mn.reshape(B), mx.reshape(B), top_p,
                     s1.reshape(B), w1.reshape(B), s2.reshape(B)).reshape(B, 16)
    b1f = params[:, 0:1]
    b2f = params[:, 1:2]
    z2 = params[:, 2:3]
    ids2 = _stage3(logits3d, mn, mx, s1, w1, s2, b1f, b2f, z2)
    return ids2.reshape(B)


# speedup vs baseline: 30.8582x; 1.0592x over previous
import functools

import jax
import jax.numpy as jnp
import numpy as np
from jax import lax
from jax.experimental import pallas as pl
from jax.experimental.pallas import tpu as pltpu
from jax.experimental.pallas import tpu_sc as plsc

B = 32
D = 128
V = 1000000
NB = 4096
NBF = np.float32(NB)
VT1 = 8192
VT3 = 8192
CBR = 96
TINY = np.float32(np.finfo(np.float32).tiny)
NEG_EPS = np.float32(1e-38)


def _mm_body(h_ref, t_ref, w_ref, lg_ref, mn_ref, mx_ref, rmin_ref, rmax_ref,
             *, nblk, v_len):
    i = pl.program_id(0)
    blk = lax.dot_general(h_ref[...], w_ref[...], (((1,), (1,)), ((), ())),
                          preferred_element_type=jnp.float32)
    lt = blk / t_ref[...]
    col = lax.broadcasted_iota(jnp.int32, lt.shape, 1) + i * VT1
    valid = col < v_len
    lt = jnp.where(valid, lt, -jnp.inf)
    lg_ref[...] = lt.reshape(B, VT1 // 128, 128)
    bmin = jnp.min(jnp.where(valid, lt, jnp.inf), axis=1, keepdims=True)
    bmax = jnp.max(lt, axis=1, keepdims=True)
    bmin_b = jnp.broadcast_to(bmin, (B, 128))
    bmax_b = jnp.broadcast_to(bmax, (B, 128))

    @pl.when(i == 0)
    def _():
        rmin_ref[...] = bmin_b
        rmax_ref[...] = bmax_b

    @pl.when(i > 0)
    def _():
        rmin_ref[...] = jnp.minimum(rmin_ref[...], bmin_b)
        rmax_ref[...] = jnp.maximum(rmax_ref[...], bmax_b)

    @pl.when(i == nblk - 1)
    def _():
        mn_ref[...] = jnp.min(rmin_ref[...], axis=1, keepdims=True)
        mx_ref[...] = jnp.max(rmax_ref[...], axis=1, keepdims=True)


def _stage1(hidden, temp2, w, v_real=V):
    nblk = (v_real + VT1 - 1) // VT1
    vb = nblk * (VT1 // 128)
    return pl.pallas_call(
        functools.partial(_mm_body, nblk=nblk, v_len=v_real),
        grid=(nblk,),
        in_specs=[
            pl.BlockSpec((B, D), lambda i: (0, 0)),
            pl.BlockSpec((B, 1), lambda i: (0, 0)),
            pl.BlockSpec((VT1, D), lambda i: (i, 0)),
        ],
        out_specs=[
            pl.BlockSpec((B, VT1 // 128, 128), lambda i: (0, i, 0)),
            pl.BlockSpec((B, 1), lambda i: (0, 0)),
            pl.BlockSpec((B, 1), lambda i: (0, 0)),
        ],
        out_shape=[
            jax.ShapeDtypeStruct((B, vb, 128), jnp.float32),
            jax.ShapeDtypeStruct((B, 1), jnp.float32),
            jax.ShapeDtypeStruct((B, 1), jnp.float32),
        ],
        scratch_shapes=[
            pltpu.VMEM((B, 128), jnp.float32),
            pltpu.VMEM((B, 128), jnp.float32),
        ],
    )(hidden, temp2, w)


def _lane_scalar(vec, lane):
    sel = jnp.where(lax.iota(jnp.int32, 16) == lane, vec, -jnp.inf)
    return jnp.max(sel)


def _sc_body(lg_hbm, mn_hbm, mx_hbm, tp_hbm, s1_hbm, w1_hbm, s2_hbm, out_hbm,
             buf0, buf1, sem0, sem1, hist, merged, mn_v, mx_v, tp_v,
             s1_v, w1_v, s2_v, outbuf, *, vb):
    nch = vb // CBR
    bufs = (buf0, buf1)
    sems = (sem0, sem1)
    wid = lax.axis_index("s") * 2 + lax.axis_index("c")
    r = wid
    pltpu.sync_copy(mn_hbm, mn_v)
    pltpu.sync_copy(mx_hbm, mx_v)
    pltpu.sync_copy(tp_hbm, tp_v)
    pltpu.sync_copy(s1_hbm, s1_v)
    pltpu.sync_copy(w1_hbm, w1_v)
    pltpu.sync_copy(s2_hbm, s2_v)
    cbase = (r // 16) * 16
    lane = r % 16
    m_s = _lane_scalar(mn_v[pl.ds(cbase, 16)], lane)
    M_s = _lane_scalar(mx_v[pl.ds(cbase, 16)], lane)
    tp_s = _lane_scalar(tp_v[pl.ds(cbase, 16)], lane)
    s1_s = _lane_scalar(s1_v[pl.ds(cbase, 16)], lane)
    w1_s = _lane_scalar(w1_v[pl.ds(cbase, 16)], lane)
    s2_s = _lane_scalar(s2_v[pl.ds(cbase, 16)], lane)
    mb = jnp.full((16,), m_s, jnp.float32)
    Mb = jnp.full((16,), M_s, jnp.float32)
    s1b = jnp.full((16,), s1_s, jnp.float32)
    lanebase = lax.iota(jnp.int32, 16) * NB

    def zero_hist():
        def zloop(j, c):
            hist[pl.ds(j * 16, 16)] = jnp.zeros((16,), jnp.float32)
            return c
        lax.fori_loop(0, (16 * NB) // 16, zloop, 0)

    def merge_total():
        def mloop(cb, tot):
            acc = jnp.zeros((16,), jnp.float32)
            for l in range(16):
                acc = acc + hist[pl.ds(l * NB + cb * 16, 16)]
            merged[pl.ds(cb * 16, 16)] = acc
            return tot + jnp.sum(acc)
        return lax.fori_loop(0, NB // 16, mloop, jnp.float32(0.0))

    def walk(tpz, offset):
        def wloop(t, carry):
            found, bstar, sab, sinc, csum = carry
            cb = NB // 16 - 1 - t
            vv = merged[pl.ds(cb * 16, 16)]
            tot = jnp.sum(vv)
            pre = plsc.cumsum(vv)
            sufinc = offset + (csum + (tot - pre) + vv)
            maskv = sufinc > tpz
            cnt = jnp.sum(maskv.astype(jnp.int32))
            has = cnt > 0
            first = jnp.logical_and(has, jnp.logical_not(found))
            blocal = cnt - 1
            pre_at = _lane_scalar(pre, blocal)
            v_at = _lane_scalar(vv, blocal)
            sab_new = csum + (tot - pre_at)
            sinc_new = offset + (sab_new + v_at)
            return (jnp.logical_or(found, has),
                    jnp.where(first, cb * 16 + blocal, bstar),
                    jnp.where(first, sab_new, sab),
                    jnp.where(first, sinc_new, sinc),
                    csum + tot)
        init = (jnp.bool_(False), jnp.int32(0), jnp.float32(0.0),
                jnp.float32(1.0), jnp.float32(0.0))
        found, bstar, sab, sinc, _ = lax.fori_loop(0, NB // 16, wloop, init)
        return bstar, sab, sinc

    def stream(pass2, b1_s, lo2b, s2b):
        def start_copy(c, par):
            pltpu.make_async_copy(lg_hbm.at[r, pl.ds(c * CBR, CBR)],
                                  bufs[par], sems[par]).start()

        def process(bufp):
            def vloop(rr, _unused):
                for u in range(8):
                    vv = bufp[rr, pl.ds(u * 16, 16)]
                    e = jnp.exp(vv - Mb)
                    t1 = (vv - mb) * s1b
                    b1v = jnp.clip(t1.astype(jnp.int32), 0, NB - 1)
                    if not pass2:
                        idx = lanebase + b1v
                        plsc.addupdate_scatter(hist, [idx], e)
                    else:
                        t2 = (vv - lo2b) * s2b
                        b2v = jnp.clip(t2.astype(jnp.int32), 0, NB - 1)
                        idx = lanebase + b2v
                        selm = b1v == jnp.full((16,), b1_s, jnp.int32)
                        plsc.addupdate_scatter(hist, [idx], e, mask=selm)
                return 0
            lax.fori_loop(0, CBR, vloop, 0)

        start_copy(0, 0)
        start_copy(1, 1)

        def pair_loop(cc, _):
            for par in range(2):
                c = cc * 2 + par
                pltpu.make_async_copy(lg_hbm.at[r, pl.ds(0, CBR)],
                                      bufs[par], sems[par]).wait()
                process(bufs[par])

                @pl.when(c + 2 < nch)
                def _():
                    start_copy(c + 2, par)
            return 0
        lax.fori_loop(0, nch // 2, pair_loop, 0)

    zero_hist()
    stream(False, None, None, None)
    z_tot = merge_total()
    tpz = tp_s * z_tot
    b1, sab1, _ = walk(tpz, jnp.float32(0.0))

    b1f = b1.astype(jnp.float32)
    lo2_s = m_s + b1f * w1_s
    lo2b = jnp.full((16,), lo2_s, jnp.float32)
    s2b = jnp.full((16,), s2_s, jnp.float32)
    zero_hist()
    stream(True, b1, lo2b, s2b)
    merge_total()
    b2, _, z2 = walk(tpz, sab1)

    io16 = lax.iota(jnp.int32, 16)
    ov = jnp.where(io16 == 0, b1f,
                   jnp.where(io16 == 1, b2.astype(jnp.float32),
                             jnp.where(io16 == 2, z2, jnp.float32(0.0))))
    outbuf[0, pl.ds(0, 16)] = ov
    pltpu.sync_copy(outbuf, out_hbm.at[r])


def _stage2(logits3d, mn, mx, top_p, s1, w1, s2):
    vb = logits3d.shape[1]
    mesh = plsc.VectorSubcoreMesh(core_axis_name="c", subcore_axis_name="s")
    kern = pl.kernel(
        functools.partial(_sc_body, vb=vb),
        out_type=jax.ShapeDtypeStruct((B, 1, 16), jnp.float32),
        mesh=mesh,
        scratch_types=[
            pltpu.VMEM((CBR, 128), jnp.float32),
            pltpu.VMEM((CBR, 128), jnp.float32),
            pltpu.SemaphoreType.DMA,
            pltpu.SemaphoreType.DMA,
            pltpu.VMEM((16 * NB,), jnp.float32),
            pltpu.VMEM((NB,), jnp.float32),
            pltpu.VMEM((B,), jnp.float32),
            pltpu.VMEM((B,), jnp.float32),
            pltpu.VMEM((B,), jnp.float32),
            pltpu.VMEM((B,), jnp.float32),
            pltpu.VMEM((B,), jnp.float32),
            pltpu.VMEM((B,), jnp.float32),
            pltpu.VMEM((1, 16), jnp.float32),
        ],
        compiler_params=pltpu.CompilerParams(needs_layout_passes=False),
    )
    return kern(logits3d, mn, mx, top_p, s1, w1, s2)


_ROT = ((13, 15, 26, 6), (17, 29, 16, 24))


def _threefry_bits(j):
    k0 = jnp.uint32(0)
    k1 = jnp.uint32(42)
    k2 = jnp.uint32(0 ^ 42 ^ 0x1BD11BDA)
    ks = (k0, k1, k2)
    x0 = jnp.zeros_like(j) + ks[0]
    x1 = j + ks[1]
    for g in range(5):
        for rr in _ROT[g % 2]:
            x0 = x0 + x1
            x1 = (x1 << jnp.uint32(rr)) | (x1 >> jnp.uint32(32 - rr))
            x1 = x0 ^ x1
        x0 = x0 + ks[(g + 1) % 3]
        x1 = x1 + ks[(g + 2) % 3] + jnp.uint32(g + 1)
    return x0 ^ x1


def _smp_body(lg_ref, mn_ref, mx_ref, s1_ref, w1_ref, s2_ref,
              b1_ref, b2_ref, z2_ref, ids_ref,
              bv_ref, bi_ref, *, nblk, v_len):
    i = pl.program_id(0)
    lt = lg_ref[...].reshape(B, VT3)
    mnb = mn_ref[...]
    mxb = mx_ref[...]
    t1 = (lt - mnb) * s1_ref[...]
    bin1 = jnp.minimum(t1.astype(jnp.int32), NB - 1)
    b1f = b1_ref[...]
    b1i = b1f.astype(jnp.int32)
    lo2 = mnb + b1f * w1_ref[...]
    t2 = (lt - lo2) * s2_ref[...]
    bin2 = jnp.clip(t2.astype(jnp.int32), 0, NB - 1)
    b2i = b2_ref[...].astype(jnp.int32)
    kept = (bin1 > b1i) | ((bin1 == b1i) & (bin2 >= b2i))
    e = jnp.exp(lt - mxb)
    p2 = jnp.where(kept, e / z2_ref[...], jnp.float32(0.0))
    z = jnp.log(p2 + NEG_EPS)

    col = lax.broadcasted_iota(jnp.int32, lt.shape, 1) + i * VT3
    row = lax.broadcasted_iota(jnp.int32, lt.shape, 0)
    j = (row * v_len + col).astype(jnp.uint32)
    bits = _threefry_bits(j)
    fb = (bits >> jnp.uint32(9)) | jnp.uint32(0x3F800000)
    f = lax.bitcast_convert_type(fb, jnp.float32) - jnp.float32(1.0)
    u = jnp.maximum(TINY, f * jnp.float32(1.0) + TINY)
    g = -jnp.log(-jnp.log(u))

    s = jnp.where(col < v_len, g + z, -jnp.inf)
    bmax = jnp.max(s, axis=1, keepdims=True)
    cand = jnp.where(s == bmax, col, jnp.int32(2**31 - 1))
    bidx = jnp.min(cand, axis=1, keepdims=True)
    bmax_b = jnp.broadcast_to(bmax, (B, 128))
    bidx_b = jnp.broadcast_to(bidx, (B, 128))

    @pl.when(i == 0)
    def _():
        bv_ref[...] = bmax_b
        bi_ref[...] = bidx_b

    @pl.when(i > 0)
    def _():
        upd = bmax_b > bv_ref[...]
        bv_ref[...] = jnp.where(upd, bmax_b, bv_ref[...])
        bi_ref[...] = jnp.where(upd, bidx_b, bi_ref[...])

    @pl.when(i == nblk - 1)
    def _():
        ids_ref[...] = jnp.min(bi_ref[...], axis=1, keepdims=True)


def _stage3(logits3d, mn, mx, s1, w1, s2, b1f, b2f, z2, v_real=V):
    nblk = logits3d.shape[1] // (VT3 // 128)
    return pl.pallas_call(
        functools.partial(_smp_body, nblk=nblk, v_len=v_real),
        grid=(nblk,),
        in_specs=[pl.BlockSpec((B, VT3 // 128, 128), lambda i: (0, i, 0))] + [
            pl.BlockSpec((B, 1), lambda i: (0, 0)) for _ in range(8)],
        out_specs=pl.BlockSpec((B, 1), lambda i: (0, 0)),
        out_shape=jax.ShapeDtypeStruct((B, 1), jnp.int32),
        scratch_shapes=[
            pltpu.VMEM((B, 128), jnp.float32),
            pltpu.VMEM((B, 128), jnp.int32),
        ],
    )(logits3d, mn, mx, s1, w1, s2, b1f, b2f, z2)


def kernel(hidden_states, temperature, top_p, embd_weight):
    temp2 = temperature.reshape(B, 1)
    logits3d, mn, mx = _stage1(hidden_states, temp2, embd_weight)
    s1 = NBF / (mx - mn)
    w1 = (mx - mn) / NBF
    s2 = NBF / w1
    params = _stage2(logits3d, mn.reshape(B), mx.reshape(B), top_p,
                     s1.reshape(B), w1.reshape(B), s2.reshape(B)).reshape(B, 16)
    b1f = params[:, 0:1]
    b2f = params[:, 1:2]
    z2 = params[:, 2:3]
    ids2 = _stage3(logits3d, mn, mx, s1, w1, s2, b1f, b2f, z2)
    return ids2.reshape(B)


# speedup vs baseline: 54.4120x; 1.7633x over previous
import functools

import jax
import jax.numpy as jnp
import numpy as np
from jax import lax
from jax.experimental import pallas as pl
from jax.experimental.pallas import tpu as pltpu
from jax.experimental.pallas import tpu_sc as plsc

B = 32
D = 128
V = 1000000
NB = 4096
NBF = np.float32(NB)
VT1 = 8192
VT3 = 8192
CBR = 96
TINY = np.float32(np.finfo(np.float32).tiny)
NEG_EPS = np.float32(1e-38)


def _mm_body(h_ref, t_ref, w_ref, lg_ref, mn_ref, mx_ref, rmin_ref, rmax_ref,
             *, nblk, v_len):
    i = pl.program_id(0)
    blk = lax.dot_general(h_ref[...], w_ref[...], (((1,), (1,)), ((), ())),
                          preferred_element_type=jnp.float32)
    lt = blk / t_ref[...]
    col = lax.broadcasted_iota(jnp.int32, lt.shape, 1) + i * VT1
    valid = col < v_len
    lt = jnp.where(valid, lt, -jnp.inf)
    lg_ref[...] = lt.reshape(B, VT1 // 128, 128)
    bmin = jnp.min(jnp.where(valid, lt, jnp.inf), axis=1, keepdims=True)
    bmax = jnp.max(lt, axis=1, keepdims=True)
    bmin_b = jnp.broadcast_to(bmin, (B, 128))
    bmax_b = jnp.broadcast_to(bmax, (B, 128))

    @pl.when(i == 0)
    def _():
        rmin_ref[...] = bmin_b
        rmax_ref[...] = bmax_b

    @pl.when(i > 0)
    def _():
        rmin_ref[...] = jnp.minimum(rmin_ref[...], bmin_b)
        rmax_ref[...] = jnp.maximum(rmax_ref[...], bmax_b)

    @pl.when(i == nblk - 1)
    def _():
        mn_ref[...] = jnp.min(rmin_ref[...], axis=1, keepdims=True)
        mx_ref[...] = jnp.max(rmax_ref[...], axis=1, keepdims=True)


def _stage1(hidden, temp2, w, v_real=V):
    nblk = (v_real + VT1 - 1) // VT1
    vb = nblk * (VT1 // 128)
    return pl.pallas_call(
        functools.partial(_mm_body, nblk=nblk, v_len=v_real),
        grid=(nblk,),
        in_specs=[
            pl.BlockSpec((B, D), lambda i: (0, 0)),
            pl.BlockSpec((B, 1), lambda i: (0, 0)),
            pl.BlockSpec((VT1, D), lambda i: (i, 0)),
        ],
        out_specs=[
            pl.BlockSpec((B, VT1 // 128, 128), lambda i: (0, i, 0)),
            pl.BlockSpec((B, 1), lambda i: (0, 0)),
            pl.BlockSpec((B, 1), lambda i: (0, 0)),
        ],
        out_shape=[
            jax.ShapeDtypeStruct((B, vb, 128), jnp.float32),
            jax.ShapeDtypeStruct((B, 1), jnp.float32),
            jax.ShapeDtypeStruct((B, 1), jnp.float32),
        ],
        scratch_shapes=[
            pltpu.VMEM((B, 128), jnp.float32),
            pltpu.VMEM((B, 128), jnp.float32),
        ],
    )(hidden, temp2, w)


def _lane_scalar(vec, lane):
    sel = jnp.where(lax.iota(jnp.int32, 16) == lane, vec, -jnp.inf)
    return jnp.max(sel)


def _sc_body(lg_hbm, mn_hbm, mx_hbm, tp_hbm, s1_hbm, w1_hbm, s2_hbm, out_hbm,
             buf0, buf1, sem0, sem1, hist, merged, mn_v, mx_v, tp_v,
             s1_v, w1_v, s2_v, outbuf, *, vb):
    nch = vb // CBR
    bufs = (buf0, buf1)
    sems = (sem0, sem1)
    wid = lax.axis_index("s") * 2 + lax.axis_index("c")
    r = wid
    pltpu.sync_copy(mn_hbm, mn_v)
    pltpu.sync_copy(mx_hbm, mx_v)
    pltpu.sync_copy(tp_hbm, tp_v)
    pltpu.sync_copy(s1_hbm, s1_v)
    pltpu.sync_copy(w1_hbm, w1_v)
    pltpu.sync_copy(s2_hbm, s2_v)
    cbase = (r // 16) * 16
    lane = r % 16
    m_s = _lane_scalar(mn_v[pl.ds(cbase, 16)], lane)
    M_s = _lane_scalar(mx_v[pl.ds(cbase, 16)], lane)
    tp_s = _lane_scalar(tp_v[pl.ds(cbase, 16)], lane)
    s1_s = _lane_scalar(s1_v[pl.ds(cbase, 16)], lane)
    w1_s = _lane_scalar(w1_v[pl.ds(cbase, 16)], lane)
    s2_s = _lane_scalar(s2_v[pl.ds(cbase, 16)], lane)
    mb = jnp.full((16,), m_s, jnp.float32)
    Mb = jnp.full((16,), M_s, jnp.float32)
    s1b = jnp.full((16,), s1_s, jnp.float32)
    lanebase = lax.iota(jnp.int32, 16) * NB

    def zero_hist():
        def zloop(j, c):
            hist[pl.ds(j * 16, 16)] = jnp.zeros((16,), jnp.float32)
            return c
        lax.fori_loop(0, (16 * NB) // 16, zloop, 0)

    def merge_total():
        def mloop(cb, tot):
            acc = jnp.zeros((16,), jnp.float32)
            for l in range(16):
                acc = acc + hist[pl.ds(l * NB + cb * 16, 16)]
            merged[pl.ds(cb * 16, 16)] = acc
            return tot + jnp.sum(acc)
        return lax.fori_loop(0, NB // 16, mloop, jnp.float32(0.0))

    def walk(tpz, offset):
        def wloop(t, carry):
            found, bstar, sab, sinc, csum = carry
            cb = NB // 16 - 1 - t
            vv = merged[pl.ds(cb * 16, 16)]
            tot = jnp.sum(vv)
            pre = plsc.cumsum(vv)
            sufinc = offset + (csum + (tot - pre) + vv)
            maskv = sufinc > tpz
            cnt = jnp.sum(maskv.astype(jnp.int32))
            has = cnt > 0
            first = jnp.logical_and(has, jnp.logical_not(found))
            blocal = cnt - 1
            pre_at = _lane_scalar(pre, blocal)
            v_at = _lane_scalar(vv, blocal)
            sab_new = csum + (tot - pre_at)
            sinc_new = offset + (sab_new + v_at)
            return (jnp.logical_or(found, has),
                    jnp.where(first, cb * 16 + blocal, bstar),
                    jnp.where(first, sab_new, sab),
                    jnp.where(first, sinc_new, sinc),
                    csum + tot)
        init = (jnp.bool_(False), jnp.int32(0), jnp.float32(0.0),
                jnp.float32(1.0), jnp.float32(0.0))
        found, bstar, sab, sinc, _ = lax.fori_loop(0, NB // 16, wloop, init)
        return bstar, sab, sinc

    def stream(pass2, b1_s, lo2b, s2b):
        def start_copy(c, par):
            pltpu.make_async_copy(lg_hbm.at[r, pl.ds(c * CBR, CBR)],
                                  bufs[par], sems[par]).start()

        def process(bufp):
            @plsc.parallel_loop(0, CBR * 8, 1, unroll=8)
            def _vloop(q):
                rr = q >> 3
                u = q & 7
                vv = bufp[rr, pl.ds(u * 16, 16)]
                e = jnp.exp(vv - Mb)
                t1 = (vv - mb) * s1b
                b1v = jnp.clip(t1.astype(jnp.int32), 0, NB - 1)
                if not pass2:
                    idx = lanebase + b1v
                    plsc.addupdate_scatter(hist, [idx], e)
                else:
                    t2 = (vv - lo2b) * s2b
                    b2v = jnp.clip(t2.astype(jnp.int32), 0, NB - 1)
                    idx = lanebase + b2v
                    selm = b1v == jnp.full((16,), b1_s, jnp.int32)
                    plsc.addupdate_scatter(hist, [idx], e, mask=selm)

        start_copy(0, 0)
        start_copy(1, 1)

        def pair_loop(cc, _):
            for par in range(2):
                c = cc * 2 + par
                pltpu.make_async_copy(lg_hbm.at[r, pl.ds(0, CBR)],
                                      bufs[par], sems[par]).wait()
                process(bufs[par])

                @pl.when(c + 2 < nch)
                def _():
                    start_copy(c + 2, par)
            return 0
        lax.fori_loop(0, nch // 2, pair_loop, 0)

    zero_hist()
    stream(False, None, None, None)
    z_tot = merge_total()
    tpz = tp_s * z_tot
    b1, sab1, _ = walk(tpz, jnp.float32(0.0))

    b1f = b1.astype(jnp.float32)
    lo2_s = m_s + b1f * w1_s
    lo2b = jnp.full((16,), lo2_s, jnp.float32)
    s2b = jnp.full((16,), s2_s, jnp.float32)
    zero_hist()
    stream(True, b1, lo2b, s2b)
    merge_total()
    b2, _, z2 = walk(tpz, sab1)

    io16 = lax.iota(jnp.int32, 16)
    ov = jnp.where(io16 == 0, b1f,
                   jnp.where(io16 == 1, b2.astype(jnp.float32),
                             jnp.where(io16 == 2, z2, jnp.float32(0.0))))
    outbuf[0, pl.ds(0, 16)] = ov
    pltpu.sync_copy(outbuf, out_hbm.at[r])


def _stage2(logits3d, mn, mx, top_p, s1, w1, s2):
    vb = logits3d.shape[1]
    mesh = plsc.VectorSubcoreMesh(core_axis_name="c", subcore_axis_name="s")
    kern = pl.kernel(
        functools.partial(_sc_body, vb=vb),
        out_type=jax.ShapeDtypeStruct((B, 1, 16), jnp.float32),
        mesh=mesh,
        scratch_types=[
            pltpu.VMEM((CBR, 128), jnp.float32),
            pltpu.VMEM((CBR, 128), jnp.float32),
            pltpu.SemaphoreType.DMA,
            pltpu.SemaphoreType.DMA,
            pltpu.VMEM((16 * NB,), jnp.float32),
            pltpu.VMEM((NB,), jnp.float32),
            pltpu.VMEM((B,), jnp.float32),
            pltpu.VMEM((B,), jnp.float32),
            pltpu.VMEM((B,), jnp.float32),
            pltpu.VMEM((B,), jnp.float32),
            pltpu.VMEM((B,), jnp.float32),
            pltpu.VMEM((B,), jnp.float32),
            pltpu.VMEM((1, 16), jnp.float32),
        ],
        compiler_params=pltpu.CompilerParams(needs_layout_passes=False),
    )
    return kern(logits3d, mn, mx, top_p, s1, w1, s2)


_ROT = ((13, 15, 26, 6), (17, 29, 16, 24))


def _threefry_bits(j):
    k0 = jnp.uint32(0)
    k1 = jnp.uint32(42)
    k2 = jnp.uint32(0 ^ 42 ^ 0x1BD11BDA)
    ks = (k0, k1, k2)
    x0 = jnp.zeros_like(j) + ks[0]
    x1 = j + ks[1]
    for g in range(5):
        for rr in _ROT[g % 2]:
            x0 = x0 + x1
            x1 = (x1 << jnp.uint32(rr)) | (x1 >> jnp.uint32(32 - rr))
            x1 = x0 ^ x1
        x0 = x0 + ks[(g + 1) % 3]
        x1 = x1 + ks[(g + 2) % 3] + jnp.uint32(g + 1)
    return x0 ^ x1


def _smp_body(lg_ref, mn_ref, mx_ref, s1_ref, w1_ref, s2_ref,
              b1_ref, b2_ref, z2_ref, ids_ref,
              bv_ref, bi_ref, *, nblk, v_len):
    i = pl.program_id(0)
    lt = lg_ref[...].reshape(B, VT3)
    mnb = mn_ref[...]
    mxb = mx_ref[...]
    t1 = (lt - mnb) * s1_ref[...]
    bin1 = jnp.minimum(t1.astype(jnp.int32), NB - 1)
    b1f = b1_ref[...]
    b1i = b1f.astype(jnp.int32)
    lo2 = mnb + b1f * w1_ref[...]
    t2 = (lt - lo2) * s2_ref[...]
    bin2 = jnp.clip(t2.astype(jnp.int32), 0, NB - 1)
    b2i = b2_ref[...].astype(jnp.int32)
    kept = (bin1 > b1i) | ((bin1 == b1i) & (bin2 >= b2i))
    e = jnp.exp(lt - mxb)
    p2 = jnp.where(kept, e / z2_ref[...], jnp.float32(0.0))
    z = jnp.log(p2 + NEG_EPS)

    col = lax.broadcasted_iota(jnp.int32, lt.shape, 1) + i * VT3
    row = lax.broadcasted_iota(jnp.int32, lt.shape, 0)
    j = (row * v_len + col).astype(jnp.uint32)
    bits = _threefry_bits(j)
    fb = (bits >> jnp.uint32(9)) | jnp.uint32(0x3F800000)
    f = lax.bitcast_convert_type(fb, jnp.float32) - jnp.float32(1.0)
    u = jnp.maximum(TINY, f * jnp.float32(1.0) + TINY)
    g = -jnp.log(-jnp.log(u))

    s = jnp.where(col < v_len, g + z, -jnp.inf)
    bmax = jnp.max(s, axis=1, keepdims=True)
    cand = jnp.where(s == bmax, col, jnp.int32(2**31 - 1))
    bidx = jnp.min(cand, axis=1, keepdims=True)
    bmax_b = jnp.broadcast_to(bmax, (B, 128))
    bidx_b = jnp.broadcast_to(bidx, (B, 128))

    @pl.when(i == 0)
    def _():
        bv_ref[...] = bmax_b
        bi_ref[...] = bidx_b

    @pl.when(i > 0)
    def _():
        upd = bmax_b > bv_ref[...]
        bv_ref[...] = jnp.where(upd, bmax_b, bv_ref[...])
        bi_ref[...] = jnp.where(upd, bidx_b, bi_ref[...])

    @pl.when(i == nblk - 1)
    def _():
        ids_ref[...] = jnp.min(bi_ref[...], axis=1, keepdims=True)


def _stage3(logits3d, mn, mx, s1, w1, s2, b1f, b2f, z2, v_real=V):
    nblk = logits3d.shape[1] // (VT3 // 128)
    return pl.pallas_call(
        functools.partial(_smp_body, nblk=nblk, v_len=v_real),
        grid=(nblk,),
        in_specs=[pl.BlockSpec((B, VT3 // 128, 128), lambda i: (0, i, 0))] + [
            pl.BlockSpec((B, 1), lambda i: (0, 0)) for _ in range(8)],
        out_specs=pl.BlockSpec((B, 1), lambda i: (0, 0)),
        out_shape=jax.ShapeDtypeStruct((B, 1), jnp.int32),
        scratch_shapes=[
            pltpu.VMEM((B, 128), jnp.float32),
            pltpu.VMEM((B, 128), jnp.int32),
        ],
    )(logits3d, mn, mx, s1, w1, s2, b1f, b2f, z2)


def kernel(hidden_states, temperature, top_p, embd_weight):
    temp2 = temperature.reshape(B, 1)
    logits3d, mn, mx = _stage1(hidden_states, temp2, embd_weight)
    s1 = NBF / (mx - mn)
    w1 = (mx - mn) / NBF
    s2 = NBF / w1
    params = _stage2(logits3d, mn.reshape(B), mx.reshape(B), top_p,
                     s1.reshape(B), w1.reshape(B), s2.reshape(B)).reshape(B, 16)
    b1f = params[:, 0:1]
    b2f = params[:, 1:2]
    z2 = params[:, 2:3]
    ids2 = _stage3(logits3d, mn, mx, s1, w1, s2, b1f, b2f, z2)
    return ids2.reshape(B)


# speedup vs baseline: 89.3275x; 1.6417x over previous
import functools

import jax
import jax.numpy as jnp
import numpy as np
from jax import lax
from jax.experimental import pallas as pl
from jax.experimental.pallas import tpu as pltpu
from jax.experimental.pallas import tpu_sc as plsc

B = 32
D = 128
V = 1000000
NB = 4096
NBF = np.float32(NB)
VT1 = 8192
VT3 = 8192
CBR = 96
TINY = np.float32(np.finfo(np.float32).tiny)
NEG_EPS = np.float32(1e-38)


def _mm_body(h_ref, t_ref, w_ref, lg_ref, mn_ref, mx_ref, rmin_ref, rmax_ref,
             *, nblk, v_len):
    i = pl.program_id(0)
    blk = lax.dot_general(h_ref[...], w_ref[...], (((1,), (1,)), ((), ())),
                          preferred_element_type=jnp.float32)
    lt = blk / t_ref[...]
    col = lax.broadcasted_iota(jnp.int32, lt.shape, 1) + i * VT1
    valid = col < v_len
    lt = jnp.where(valid, lt, -jnp.inf)
    lg_ref[...] = lt.reshape(B, VT1 // 128, 128)
    bmin = jnp.min(jnp.where(valid, lt, jnp.inf), axis=1, keepdims=True)
    bmax = jnp.max(lt, axis=1, keepdims=True)
    bmin_b = jnp.broadcast_to(bmin, (B, 128))
    bmax_b = jnp.broadcast_to(bmax, (B, 128))

    @pl.when(i == 0)
    def _():
        rmin_ref[...] = bmin_b
        rmax_ref[...] = bmax_b

    @pl.when(i > 0)
    def _():
        rmin_ref[...] = jnp.minimum(rmin_ref[...], bmin_b)
        rmax_ref[...] = jnp.maximum(rmax_ref[...], bmax_b)

    @pl.when(i == nblk - 1)
    def _():
        mn_ref[...] = jnp.min(rmin_ref[...], axis=1, keepdims=True)
        mx_ref[...] = jnp.max(rmax_ref[...], axis=1, keepdims=True)


def _stage1(hidden, temp2, w, v_real=V):
    nblk = (v_real + VT1 - 1) // VT1
    vb = nblk * (VT1 // 128)
    return pl.pallas_call(
        functools.partial(_mm_body, nblk=nblk, v_len=v_real),
        grid=(nblk,),
        in_specs=[
            pl.BlockSpec((B, D), lambda i: (0, 0)),
            pl.BlockSpec((B, 1), lambda i: (0, 0)),
            pl.BlockSpec((VT1, D), lambda i: (i, 0)),
        ],
        out_specs=[
            pl.BlockSpec((B, VT1 // 128, 128), lambda i: (0, i, 0)),
            pl.BlockSpec((B, 1), lambda i: (0, 0)),
            pl.BlockSpec((B, 1), lambda i: (0, 0)),
        ],
        out_shape=[
            jax.ShapeDtypeStruct((B, vb, 128), jnp.float32),
            jax.ShapeDtypeStruct((B, 1), jnp.float32),
            jax.ShapeDtypeStruct((B, 1), jnp.float32),
        ],
        scratch_shapes=[
            pltpu.VMEM((B, 128), jnp.float32),
            pltpu.VMEM((B, 128), jnp.float32),
        ],
    )(hidden, temp2, w)


def _lane_scalar(vec, lane):
    sel = jnp.where(lax.iota(jnp.int32, 16) == lane, vec, -jnp.inf)
    return jnp.max(sel)


def _sc_body(lg_hbm, mn_hbm, mx_hbm, tp_hbm, s1_hbm, w1_hbm, s2_hbm, out_hbm,
             buf0, buf1, sem0, sem1, hist, merged, mn_v, mx_v, tp_v,
             s1_v, w1_v, s2_v, outbuf, *, vb):
    nch = vb // CBR
    bufs = (buf0, buf1)
    sems = (sem0, sem1)
    wid = lax.axis_index("s") * 2 + lax.axis_index("c")
    r = wid
    pltpu.sync_copy(mn_hbm, mn_v)
    pltpu.sync_copy(mx_hbm, mx_v)
    pltpu.sync_copy(tp_hbm, tp_v)
    pltpu.sync_copy(s1_hbm, s1_v)
    pltpu.sync_copy(w1_hbm, w1_v)
    pltpu.sync_copy(s2_hbm, s2_v)
    cbase = (r // 16) * 16
    lane = r % 16
    m_s = _lane_scalar(mn_v[pl.ds(cbase, 16)], lane)
    M_s = _lane_scalar(mx_v[pl.ds(cbase, 16)], lane)
    tp_s = _lane_scalar(tp_v[pl.ds(cbase, 16)], lane)
    s1_s = _lane_scalar(s1_v[pl.ds(cbase, 16)], lane)
    w1_s = _lane_scalar(w1_v[pl.ds(cbase, 16)], lane)
    s2_s = _lane_scalar(s2_v[pl.ds(cbase, 16)], lane)
    mb = jnp.full((16,), m_s, jnp.float32)
    Mb = jnp.full((16,), M_s, jnp.float32)
    s1b = jnp.full((16,), s1_s, jnp.float32)
    lanebase = lax.iota(jnp.int32, 16) * NB

    def zero_hist():
        def zloop(j, c):
            hist[pl.ds(j * 16, 16)] = jnp.zeros((16,), jnp.float32)
            return c
        lax.fori_loop(0, (16 * NB) // 16, zloop, 0)

    def merge_total():
        def mloop(cb, tot):
            acc = jnp.zeros((16,), jnp.float32)
            for l in range(16):
                acc = acc + hist[pl.ds(l * NB + cb * 16, 16)]
            merged[pl.ds(cb * 16, 16)] = acc
            return tot + jnp.sum(acc)
        return lax.fori_loop(0, NB // 16, mloop, jnp.float32(0.0))

    def walk(tpz, offset):
        def wloop(t, carry):
            found, bstar, sab, sinc, csum = carry
            cb = NB // 16 - 1 - t
            vv = merged[pl.ds(cb * 16, 16)]
            tot = jnp.sum(vv)
            pre = plsc.cumsum(vv)
            sufinc = offset + (csum + (tot - pre) + vv)
            maskv = sufinc > tpz
            cnt = jnp.sum(maskv.astype(jnp.int32))
            has = cnt > 0
            first = jnp.logical_and(has, jnp.logical_not(found))
            blocal = cnt - 1
            pre_at = _lane_scalar(pre, blocal)
            v_at = _lane_scalar(vv, blocal)
            sab_new = csum + (tot - pre_at)
            sinc_new = offset + (sab_new + v_at)
            return (jnp.logical_or(found, has),
                    jnp.where(first, cb * 16 + blocal, bstar),
                    jnp.where(first, sab_new, sab),
                    jnp.where(first, sinc_new, sinc),
                    csum + tot)
        init = (jnp.bool_(False), jnp.int32(0), jnp.float32(0.0),
                jnp.float32(1.0), jnp.float32(0.0))
        found, bstar, sab, sinc, _ = lax.fori_loop(0, NB // 16, wloop, init)
        return bstar, sab, sinc

    def stream(pass2, b1_s, lo2b, s2b):
        def start_copy(c, par):
            pltpu.make_async_copy(lg_hbm.at[r, pl.ds(c * CBR, CBR)],
                                  bufs[par], sems[par]).start()

        def process(bufp):
            @plsc.parallel_loop(0, CBR * 8, 1, unroll=8)
            def _vloop(q):
                rr = q >> 3
                u = q & 7
                vv = bufp[rr, pl.ds(u * 16, 16)]
                e = jnp.exp(vv - Mb)
                t1 = (vv - mb) * s1b
                b1v = jnp.clip(t1.astype(jnp.int32), 0, NB - 1)
                if not pass2:
                    idx = lanebase + b1v
                    plsc.addupdate_scatter(hist, [idx], e)
                else:
                    t2 = (vv - lo2b) * s2b
                    b2v = jnp.clip(t2.astype(jnp.int32), 0, NB - 1)
                    idx = lanebase + b2v
                    selm = b1v == jnp.full((16,), b1_s, jnp.int32)
                    plsc.addupdate_scatter(hist, [idx], e, mask=selm)

        start_copy(0, 0)
        start_copy(1, 1)

        def pair_loop(cc, _):
            for par in range(2):
                c = cc * 2 + par
                pltpu.make_async_copy(lg_hbm.at[r, pl.ds(0, CBR)],
                                      bufs[par], sems[par]).wait()
                process(bufs[par])

                @pl.when(c + 2 < nch)
                def _():
                    start_copy(c + 2, par)
            return 0
        lax.fori_loop(0, nch // 2, pair_loop, 0)

    zero_hist()
    stream(False, None, None, None)
    z_tot = merge_total()
    tpz = tp_s * z_tot
    b1, sab1, _ = walk(tpz, jnp.float32(0.0))

    b1f = b1.astype(jnp.float32)
    lo2_s = m_s + b1f * w1_s
    lo2b = jnp.full((16,), lo2_s, jnp.float32)
    s2b = jnp.full((16,), s2_s, jnp.float32)
    zero_hist()
    stream(True, b1, lo2b, s2b)
    merge_total()
    b2, _, z2 = walk(tpz, sab1)

    io16 = lax.iota(jnp.int32, 16)
    ov = jnp.where(io16 == 0, b1f,
                   jnp.where(io16 == 1, b2.astype(jnp.float32),
                             jnp.where(io16 == 2, z2, jnp.float32(0.0))))
    outbuf[0, pl.ds(0, 16)] = ov
    pltpu.sync_copy(outbuf, out_hbm.at[r])


def _stage2(logits3d, mn, mx, top_p, s1, w1, s2):
    vb = logits3d.shape[1]
    mesh = plsc.VectorSubcoreMesh(core_axis_name="c", subcore_axis_name="s")
    kern = pl.kernel(
        functools.partial(_sc_body, vb=vb),
        out_type=jax.ShapeDtypeStruct((B, 1, 16), jnp.float32),
        mesh=mesh,
        scratch_types=[
            pltpu.VMEM((CBR, 128), jnp.float32),
            pltpu.VMEM((CBR, 128), jnp.float32),
            pltpu.SemaphoreType.DMA,
            pltpu.SemaphoreType.DMA,
            pltpu.VMEM((16 * NB,), jnp.float32),
            pltpu.VMEM((NB,), jnp.float32),
            pltpu.VMEM((B,), jnp.float32),
            pltpu.VMEM((B,), jnp.float32),
            pltpu.VMEM((B,), jnp.float32),
            pltpu.VMEM((B,), jnp.float32),
            pltpu.VMEM((B,), jnp.float32),
            pltpu.VMEM((B,), jnp.float32),
            pltpu.VMEM((1, 16), jnp.float32),
        ],
        compiler_params=pltpu.CompilerParams(needs_layout_passes=False),
    )
    return kern(logits3d, mn, mx, top_p, s1, w1, s2)


_ROT = ((13, 15, 26, 6), (17, 29, 16, 24))


def _threefry_bits(j):
    k0 = jnp.uint32(0)
    k1 = jnp.uint32(42)
    k2 = jnp.uint32(0 ^ 42 ^ 0x1BD11BDA)
    ks = (k0, k1, k2)
    x0 = jnp.zeros_like(j) + ks[0]
    x1 = j + ks[1]
    for g in range(5):
        for rr in _ROT[g % 2]:
            x0 = x0 + x1
            x1 = (x1 << jnp.uint32(rr)) | (x1 >> jnp.uint32(32 - rr))
            x1 = x0 ^ x1
        x0 = x0 + ks[(g + 1) % 3]
        x1 = x1 + ks[(g + 2) % 3] + jnp.uint32(g + 1)
    return x0 ^ x1


def _gum_body(g_ref, *, v_len):
    i = pl.program_id(0)
    shape = (B, VT3)
    col = lax.broadcasted_iota(jnp.int32, shape, 1) + i * VT3
    row = lax.broadcasted_iota(jnp.int32, shape, 0)
    j = (row * v_len + col).astype(jnp.uint32)
    bits = _threefry_bits(j)
    fb = (bits >> jnp.uint32(9)) | jnp.uint32(0x3F800000)
    f = lax.bitcast_convert_type(fb, jnp.float32) - jnp.float32(1.0)
    u = jnp.maximum(TINY, f * jnp.float32(1.0) + TINY)
    g = -jnp.log(-jnp.log(u))
    g_ref[...] = g.reshape(B, VT3 // 128, 128)


def _stage_gum(vb, v_real=V):
    nblk = vb // (VT3 // 128)
    return pl.pallas_call(
        functools.partial(_gum_body, v_len=v_real),
        grid=(nblk,),
        in_specs=[],
        out_specs=pl.BlockSpec((B, VT3 // 128, 128), lambda i: (0, i, 0)),
        out_shape=jax.ShapeDtypeStruct((B, vb, 128), jnp.float32),
    )()


def _smp_body(lg_ref, g_ref, mn_ref, mx_ref, s1_ref, w1_ref, s2_ref,
              b1_ref, b2_ref, z2_ref, ids_ref,
              bv_ref, bi_ref, *, nblk, v_len):
    i = pl.program_id(0)
    lt = lg_ref[...].reshape(B, VT3)
    mnb = mn_ref[...]
    mxb = mx_ref[...]
    t1 = (lt - mnb) * s1_ref[...]
    bin1 = jnp.minimum(t1.astype(jnp.int32), NB - 1)
    b1f = b1_ref[...]
    b1i = b1f.astype(jnp.int32)
    lo2 = mnb + b1f * w1_ref[...]
    t2 = (lt - lo2) * s2_ref[...]
    bin2 = jnp.clip(t2.astype(jnp.int32), 0, NB - 1)
    b2i = b2_ref[...].astype(jnp.int32)
    kept = (bin1 > b1i) | ((bin1 == b1i) & (bin2 >= b2i))
    e = jnp.exp(lt - mxb)
    p2 = jnp.where(kept, e / z2_ref[...], jnp.float32(0.0))
    z = jnp.log(p2 + NEG_EPS)

    col = lax.broadcasted_iota(jnp.int32, lt.shape, 1) + i * VT3
    g = g_ref[...].reshape(B, VT3)
    s = jnp.where(col < v_len, g + z, -jnp.inf)
    bmax = jnp.max(s, axis=1, keepdims=True)
    cand = jnp.where(s == bmax, col, jnp.int32(2**31 - 1))
    bidx = jnp.min(cand, axis=1, keepdims=True)
    bmax_b = jnp.broadcast_to(bmax, (B, 128))
    bidx_b = jnp.broadcast_to(bidx, (B, 128))

    @pl.when(i == 0)
    def _():
        bv_ref[...] = bmax_b
        bi_ref[...] = bidx_b

    @pl.when(i > 0)
    def _():
        upd = bmax_b > bv_ref[...]
        bv_ref[...] = jnp.where(upd, bmax_b, bv_ref[...])
        bi_ref[...] = jnp.where(upd, bidx_b, bi_ref[...])

    @pl.when(i == nblk - 1)
    def _():
        ids_ref[...] = jnp.min(bi_ref[...], axis=1, keepdims=True)


def _stage3(logits3d, gum3d, mn, mx, s1, w1, s2, b1f, b2f, z2, v_real=V):
    nblk = logits3d.shape[1] // (VT3 // 128)
    return pl.pallas_call(
        functools.partial(_smp_body, nblk=nblk, v_len=v_real),
        grid=(nblk,),
        in_specs=[pl.BlockSpec((B, VT3 // 128, 128), lambda i: (0, i, 0)),
                  pl.BlockSpec((B, VT3 // 128, 128), lambda i: (0, i, 0))] + [
            pl.BlockSpec((B, 1), lambda i: (0, 0)) for _ in range(8)],
        out_specs=pl.BlockSpec((B, 1), lambda i: (0, 0)),
        out_shape=jax.ShapeDtypeStruct((B, 1), jnp.int32),
        scratch_shapes=[
            pltpu.VMEM((B, 128), jnp.float32),
            pltpu.VMEM((B, 128), jnp.int32),
        ],
    )(logits3d, gum3d, mn, mx, s1, w1, s2, b1f, b2f, z2)


def kernel(hidden_states, temperature, top_p, embd_weight):
    temp2 = temperature.reshape(B, 1)
    logits3d, mn, mx = _stage1(hidden_states, temp2, embd_weight)
    s1 = NBF / (mx - mn)
    w1 = (mx - mn) / NBF
    s2 = NBF / w1
    gum3d = _stage_gum(logits3d.shape[1])
    params = _stage2(logits3d, mn.reshape(B), mx.reshape(B), top_p,
                     s1.reshape(B), w1.reshape(B), s2.reshape(B)).reshape(B, 16)
    b1f = params[:, 0:1]
    b2f = params[:, 1:2]
    z2 = params[:, 2:3]
    ids2 = _stage3(logits3d, gum3d, mn, mx, s1, w1, s2, b1f, b2f, z2)
    return ids2.reshape(B)
